# Initial kernel scaffold; baseline (speedup 1.0000x reference)
import jax
import jax.numpy as jnp
from jax.experimental import pallas as pl


def kernel(x, A_pos, A_neg, W_in, b_in, W_pos0, b_pos0, W_neg0, b_neg0, W_pos1, b_pos1, W_neg1, b_neg1):
    raise NotImplementedError("write your pallas kernel here")


# speedup vs baseline: 1.1100x; 1.1100x over previous
import jax
import jax.numpy as jnp
from jax.experimental import pallas as pl

N = 4096
H = 256
BM = 512


def _prep_kernel(x_ref, WinT_ref, b_ref, h_ref):
    h_ref[...] = jnp.tanh(
        jnp.dot(x_ref[...], WinT_ref[...], preferred_element_type=jnp.float32)
        + b_ref[...]
    )


def _layer_kernel(Ap_ref, An_ref, h_ref, WpT_ref, WnT_ref, bp_ref, bn_ref,
                  out_ref):
    hp = jnp.dot(Ap_ref[...], h_ref[...], preferred_element_type=jnp.float32)
    hn = jnp.dot(An_ref[...], h_ref[...], preferred_element_type=jnp.float32)
    out_ref[...] = jnp.tanh(
        jnp.dot(hp, WpT_ref[...], preferred_element_type=jnp.float32)
        + bp_ref[...]
        + jnp.dot(hn, WnT_ref[...], preferred_element_type=jnp.float32)
        + bn_ref[...]
    )


def _full_spec(shape):
    return pl.BlockSpec(shape, lambda i: (0,) * len(shape))


def _rows_spec(width):
    return pl.BlockSpec((BM, width), lambda i: (i, 0))


def _layer(A_pos, A_neg, h, WpT, WnT, bp, bn):
    return pl.pallas_call(
        _layer_kernel,
        grid=(N // BM,),
        in_specs=[
            _rows_spec(N),
            _rows_spec(N),
            _full_spec((N, H)),
            _full_spec((H, H)),
            _full_spec((H, H)),
            _full_spec((1, H)),
            _full_spec((1, H)),
        ],
        out_specs=_rows_spec(H),
        out_shape=jax.ShapeDtypeStruct((N, H), jnp.float32),
    )(A_pos, A_neg, h, WpT, WnT, bp, bn)


@jax.jit
def kernel(x, A_pos, A_neg, W_in, b_in, W_pos0, b_pos0, W_neg0, b_neg0,
           W_pos1, b_pos1, W_neg1, b_neg1):
    h = pl.pallas_call(
        _prep_kernel,
        grid=(N // BM,),
        in_specs=[
            _rows_spec(H),
            _full_spec((H, H)),
            _full_spec((1, H)),
        ],
        out_specs=_rows_spec(H),
        out_shape=jax.ShapeDtypeStruct((N, H), jnp.float32),
    )(x, W_in.T, b_in.reshape(1, H))

    h = _layer(A_pos, A_neg, h, W_pos0.T, W_neg0.T,
               b_pos0.reshape(1, H), b_neg0.reshape(1, H))
    h = _layer(A_pos, A_neg, h, W_pos1.T, W_neg1.T,
               b_pos1.reshape(1, H), b_neg1.reshape(1, H))
    return h


# speedup vs baseline: 1.1380x; 1.0252x over previous
import jax
import jax.numpy as jnp
from jax.experimental import pallas as pl
from jax.experimental.pallas import tpu as pltpu

N = 4096
H = 256
BM = 512
NB = N // BM


def _gcn_kernel(x_ref, Ap_ref, An_ref, WinT_ref, bin_ref,
                Wp0T_ref, Wn0T_ref, b0_ref,
                Wp1T_ref, Wn1T_ref, b1_ref,
                out_ref, h0_ref, h1_ref):
    s = pl.program_id(0)

    @pl.when(s == 0)
    def _prep():
        h0_ref[...] = jnp.tanh(
            jnp.dot(x_ref[...], WinT_ref[...],
                    preferred_element_type=jnp.float32)
            + bin_ref[...]
        )

    def stripe(h, WpT, WnT, b):
        hp = jnp.dot(Ap_ref[...], h, preferred_element_type=jnp.float32)
        hn = jnp.dot(An_ref[...], h, preferred_element_type=jnp.float32)
        return jnp.tanh(
            jnp.dot(hp, WpT, preferred_element_type=jnp.float32)
            + jnp.dot(hn, WnT, preferred_element_type=jnp.float32)
            + b
        )

    @pl.when(s < NB)
    def _layer0():
        t = stripe(h0_ref[...], Wp0T_ref[...], Wn0T_ref[...], b0_ref[...])
        h1_ref[pl.ds(s * BM, BM), :] = t

    @pl.when(s >= NB)
    def _layer1():
        out_ref[...] = stripe(h1_ref[...], Wp1T_ref[...], Wn1T_ref[...],
                              b1_ref[...])


def _stripe_spec(width):
    return pl.BlockSpec((BM, width), lambda s: (s % NB, 0))


def _full_spec(shape):
    return pl.BlockSpec(shape, lambda s: (0,) * len(shape))


@jax.jit
def kernel(x, A_pos, A_neg, W_in, b_in, W_pos0, b_pos0, W_neg0, b_neg0,
           W_pos1, b_pos1, W_neg1, b_neg1):
    f32 = jnp.float32
    return pl.pallas_call(
        _gcn_kernel,
        grid=(2 * NB,),
        in_specs=[
            _full_spec((N, H)),
            _stripe_spec(N),
            _stripe_spec(N),
            _full_spec((H, H)),
            _full_spec((1, H)),
            _full_spec((H, H)),
            _full_spec((H, H)),
            _full_spec((1, H)),
            _full_spec((H, H)),
            _full_spec((H, H)),
            _full_spec((1, H)),
        ],
        out_specs=pl.BlockSpec((BM, H),
                               lambda s: (jnp.maximum(s - NB, 0), 0)),
        out_shape=jax.ShapeDtypeStruct((N, H), f32),
        scratch_shapes=[
            pltpu.VMEM((N, H), f32),
            pltpu.VMEM((N, H), f32),
        ],
    )(x, A_pos, A_neg, W_in.T, b_in.reshape(1, H),
      W_pos0.T, W_neg0.T, (b_pos0 + b_neg0).reshape(1, H),
      W_pos1.T, W_neg1.T, (b_pos1 + b_neg1).reshape(1, H))
